# Initial kernel scaffold; baseline (speedup 1.0000x reference)
import jax
import jax.numpy as jnp
from jax.experimental import pallas as pl


def kernel(X_real, X_imag, L_real_0, L_real_1, L_imag_0, L_imag_1, R, Qreal, Qimag, weight, weight_long, weight_res, bias):
    raise NotImplementedError("write your pallas kernel here")


# speedup vs baseline: 5.3872x; 5.3872x over previous
import jax
import jax.numpy as jnp
from jax.experimental import pallas as pl
from jax.experimental.pallas import tpu as pltpu

N = 2048
IN_C = 512
OC = 512
OCP = 256
M = 128
ROWS = 256
F32 = jnp.float32


def _dot(a, b):
    return jnp.dot(a, b, preferred_element_type=F32)


def _dot_t(a, b):
    return jax.lax.dot_general(a, b, (((0,), (0,)), ((), ())),
                               preferred_element_type=F32)


def _phase_a(xr_ref, xi_ref, w01_ref, wl_ref, wres_ref, qr_ref, qi_ref,
             rcol_ref, zc0_ref, zc1_ref, uu_ref, vv_ref):
    xr = xr_ref[...]
    xi = xi_ref[...]
    w01 = w01_ref[...]
    xrw = _dot(xr, w01)
    xiw = _dot(xi, w01)
    zc0_ref[...] = jnp.concatenate([xrw[:, :OCP], xiw[:, :OCP]], axis=1)
    zc1_ref[...] = jnp.concatenate([xrw[:, OCP:], xiw[:, OCP:]], axis=1)

    qr = qr_ref[...]
    qi = qi_ref[...]
    rcol = rcol_ref[...]
    t_long = rcol * rcol
    t_res = rcol

    yr_l = _dot(xr, wl_ref[...])
    yi_l = _dot(xi, wl_ref[...])
    u_l = t_long * (_dot_t(qr, yr_l) + _dot_t(qi, yi_l))
    v_l = t_long * (_dot_t(qi, yr_l) - _dot_t(qr, yi_l))

    yr_r = _dot(xr, wres_ref[...])
    yi_r = _dot(xi, wres_ref[...])
    u_r = t_res * (_dot_t(qr, yr_r) + _dot_t(qi, yi_r))
    v_r = t_res * (_dot_t(qi, yr_r) - _dot_t(qr, yi_r))

    uu_ref[...] = jnp.concatenate([u_r[:, :OCP], u_r[:, OCP:] + u_l], axis=1)
    vv_ref[...] = jnp.concatenate([v_r[:, :OCP], v_r[:, OCP:] + v_l], axis=1)


def _phase_b(lr0_ref, li0_ref, lr1_ref, li1_ref, zc0_ref, zc1_ref,
             qr_ref, qi_ref, uu_ref, vv_ref, bias_ref,
             real_ref, imag_ref):
    i = pl.program_id(0)
    zc0 = zc0_ref[...]
    zc1 = zc1_ref[...]

    p0 = _dot(lr0_ref[...], zc0)
    q0 = _dot(li0_ref[...], zc0)
    p1 = _dot(lr1_ref[...], zc1)
    q1 = _dot(li1_ref[...], zc1)

    dense_real = (p0[:, :OCP] - q0[:, OCP:]) + (p1[:, :OCP] - q1[:, OCP:])
    dense_imag = (q0[:, :OCP] + p0[:, OCP:]) + (q1[:, :OCP] + p1[:, OCP:])

    uu = uu_ref[...]
    vv = vv_ref[...]
    qr = qr_ref[...]
    qi = qi_ref[...]
    spec_real = _dot(qr, uu) + _dot(qi, vv)
    spec_imag = _dot(qi, uu) - _dot(qr, vv)

    z0 = zc0_ref[pl.ds(i * ROWS, ROWS), :]
    z1 = zc1_ref[pl.ds(i * ROWS, ROWS), :]
    bias = bias_ref[...]

    real_left = dense_real + spec_real[:, :OCP] + z0[:, :OCP] + bias[:, :OCP]
    real_right = spec_real[:, OCP:] + z1[:, :OCP] + bias[:, OCP:]
    imag_left = dense_imag + spec_imag[:, :OCP] + z0[:, OCP:] + bias[:, :OCP]
    imag_right = spec_imag[:, OCP:] + z1[:, OCP:] + bias[:, OCP:]

    real_ref[...] = jnp.concatenate([real_left, real_right], axis=1)
    imag_ref[...] = jnp.concatenate([imag_left, imag_right], axis=1)


def kernel(X_real, X_imag, L_real_0, L_real_1, L_imag_0, L_imag_1, R,
           Qreal, Qimag, weight, weight_long, weight_res, bias):
    w01 = jnp.concatenate([weight[0], weight[1]], axis=-1)
    wl = weight_long[0]
    wres = weight_res[0]
    rcol = R.reshape(M, 1)

    full = lambda s: pl.BlockSpec(s, lambda: (0, 0))
    zc0, zc1, uu, vv = pl.pallas_call(
        _phase_a,
        out_shape=(
            jax.ShapeDtypeStruct((N, OC), F32),
            jax.ShapeDtypeStruct((N, OC), F32),
            jax.ShapeDtypeStruct((M, OC), F32),
            jax.ShapeDtypeStruct((M, OC), F32),
        ),
        in_specs=[
            full((N, IN_C)), full((N, IN_C)), full((IN_C, OC)),
            full((IN_C, OCP)), full((IN_C, OC)),
            full((N, M)), full((N, M)), full((M, 1)),
        ],
        out_specs=(full((N, OC)), full((N, OC)), full((M, OC)), full((M, OC))),
    )(X_real, X_imag, w01, wl, wres, Qreal, Qimag, rcol)

    row = pl.BlockSpec((ROWS, N), lambda i: (i, 0))
    rowq = pl.BlockSpec((ROWS, M), lambda i: (i, 0))
    whole = lambda s: pl.BlockSpec(s, lambda i: (0, 0))
    out_row = pl.BlockSpec((ROWS, OC), lambda i: (i, 0))

    real, imag = pl.pallas_call(
        _phase_b,
        grid=(N // ROWS,),
        out_shape=(
            jax.ShapeDtypeStruct((N, OC), F32),
            jax.ShapeDtypeStruct((N, OC), F32),
        ),
        in_specs=[
            row, row, row, row,
            whole((N, OC)), whole((N, OC)),
            rowq, rowq,
            whole((M, OC)), whole((M, OC)), whole((1, OC)),
        ],
        out_specs=(out_row, out_row),
        compiler_params=pltpu.CompilerParams(
            dimension_semantics=("arbitrary",)),
    )(L_real_0, L_imag_0, L_real_1, L_imag_1, zc0, zc1,
      Qreal, Qimag, uu, vv, bias)

    return (real, imag)


# speedup vs baseline: 5.5477x; 1.0298x over previous
import jax
import jax.numpy as jnp
from jax.experimental import pallas as pl
from jax.experimental.pallas import tpu as pltpu

N = 2048
IN_C = 512
OC = 512
OCP = 256
M = 128
ROWS = 256
F32 = jnp.float32


BF16 = jnp.bfloat16


def _dot(a, b):
    return jnp.dot(a.astype(BF16), b.astype(BF16), preferred_element_type=F32)


def _dot_t(a, b):
    return jax.lax.dot_general(a.astype(BF16), b.astype(BF16),
                               (((0,), (0,)), ((), ())),
                               preferred_element_type=F32)


def _phase_a(xr_ref, xi_ref, w01_ref, wl_ref, wres_ref, qr_ref, qi_ref,
             rcol_ref, zc0_ref, zc1_ref, uu_ref, vv_ref):
    xr = xr_ref[...]
    xi = xi_ref[...]
    w01 = w01_ref[...]
    xrw = _dot(xr, w01)
    xiw = _dot(xi, w01)
    zc0_ref[...] = jnp.concatenate(
        [xrw[:, :OCP], xiw[:, :OCP]], axis=1).astype(BF16)
    zc1_ref[...] = jnp.concatenate(
        [xrw[:, OCP:], xiw[:, OCP:]], axis=1).astype(BF16)

    qr = qr_ref[...]
    qi = qi_ref[...]
    rcol = rcol_ref[...]
    t_long = rcol * rcol
    t_res = rcol

    yr_l = _dot(xr, wl_ref[...])
    yi_l = _dot(xi, wl_ref[...])
    u_l = t_long * (_dot_t(qr, yr_l) + _dot_t(qi, yi_l))
    v_l = t_long * (_dot_t(qi, yr_l) - _dot_t(qr, yi_l))

    yr_r = _dot(xr, wres_ref[...])
    yi_r = _dot(xi, wres_ref[...])
    u_r = t_res * (_dot_t(qr, yr_r) + _dot_t(qi, yi_r))
    v_r = t_res * (_dot_t(qi, yr_r) - _dot_t(qr, yi_r))

    uu_ref[...] = jnp.concatenate([u_r[:, :OCP], u_r[:, OCP:] + u_l], axis=1)
    vv_ref[...] = jnp.concatenate([v_r[:, :OCP], v_r[:, OCP:] + v_l], axis=1)


def _phase_b(lr0_ref, li0_ref, lr1_ref, li1_ref, zc0_ref, zc1_ref,
             qr_ref, qi_ref, uu_ref, vv_ref, bias_ref,
             real_ref, imag_ref):
    i = pl.program_id(0)
    zc0 = zc0_ref[...]
    zc1 = zc1_ref[...]

    p0 = _dot(lr0_ref[...], zc0)
    q0 = _dot(li0_ref[...], zc0)
    p1 = _dot(lr1_ref[...], zc1)
    q1 = _dot(li1_ref[...], zc1)

    dense_real = (p0[:, :OCP] - q0[:, OCP:]) + (p1[:, :OCP] - q1[:, OCP:])
    dense_imag = (q0[:, :OCP] + p0[:, OCP:]) + (q1[:, :OCP] + p1[:, OCP:])

    uu = uu_ref[...]
    vv = vv_ref[...]
    qr = qr_ref[...]
    qi = qi_ref[...]
    spec_real = _dot(qr, uu) + _dot(qi, vv)
    spec_imag = _dot(qi, uu) - _dot(qr, vv)

    z0 = zc0_ref[pl.ds(i * ROWS, ROWS), :].astype(F32)
    z1 = zc1_ref[pl.ds(i * ROWS, ROWS), :].astype(F32)
    bias = bias_ref[...]

    real_left = dense_real + spec_real[:, :OCP] + z0[:, :OCP] + bias[:, :OCP]
    real_right = spec_real[:, OCP:] + z1[:, :OCP] + bias[:, OCP:]
    imag_left = dense_imag + spec_imag[:, :OCP] + z0[:, OCP:] + bias[:, :OCP]
    imag_right = spec_imag[:, OCP:] + z1[:, OCP:] + bias[:, OCP:]

    real_ref[...] = jnp.concatenate([real_left, real_right], axis=1)
    imag_ref[...] = jnp.concatenate([imag_left, imag_right], axis=1)


def kernel(X_real, X_imag, L_real_0, L_real_1, L_imag_0, L_imag_1, R,
           Qreal, Qimag, weight, weight_long, weight_res, bias):
    w01 = jnp.concatenate([weight[0], weight[1]], axis=-1)
    wl = weight_long[0]
    wres = weight_res[0]
    rcol = R.reshape(M, 1)

    full = lambda s: pl.BlockSpec(s, lambda: (0, 0))
    zc0, zc1, uu, vv = pl.pallas_call(
        _phase_a,
        out_shape=(
            jax.ShapeDtypeStruct((N, OC), BF16),
            jax.ShapeDtypeStruct((N, OC), BF16),
            jax.ShapeDtypeStruct((M, OC), F32),
            jax.ShapeDtypeStruct((M, OC), F32),
        ),
        in_specs=[
            full((N, IN_C)), full((N, IN_C)), full((IN_C, OC)),
            full((IN_C, OCP)), full((IN_C, OC)),
            full((N, M)), full((N, M)), full((M, 1)),
        ],
        out_specs=(full((N, OC)), full((N, OC)), full((M, OC)), full((M, OC))),
    )(X_real, X_imag, w01, wl, wres, Qreal, Qimag, rcol)

    row = pl.BlockSpec((ROWS, N), lambda i: (i, 0))
    rowq = pl.BlockSpec((ROWS, M), lambda i: (i, 0))
    whole = lambda s: pl.BlockSpec(s, lambda i: (0, 0))
    out_row = pl.BlockSpec((ROWS, OC), lambda i: (i, 0))

    real, imag = pl.pallas_call(
        _phase_b,
        grid=(N // ROWS,),
        out_shape=(
            jax.ShapeDtypeStruct((N, OC), F32),
            jax.ShapeDtypeStruct((N, OC), F32),
        ),
        in_specs=[
            row, row, row, row,
            whole((N, OC)), whole((N, OC)),
            rowq, rowq,
            whole((M, OC)), whole((M, OC)), whole((1, OC)),
        ],
        out_specs=(out_row, out_row),
        compiler_params=pltpu.CompilerParams(
            dimension_semantics=("arbitrary",)),
    )(L_real_0, L_imag_0, L_real_1, L_imag_1, zc0, zc1,
      Qreal, Qimag, uu, vv, bias)

    return (real, imag)
